# baseline (speedup 1.0000x reference)
import functools

import jax
import jax.numpy as jnp
from jax import lax
from jax.experimental import pallas as pl
from jax.experimental.pallas import tpu as pltpu
from jax.experimental.pallas import tpu_sc as plsc

VOCAB = 1000000
REL = 1000
DIM = 32
B = 16384

NC, NS = 2, 16
NW = NC * NS
RPW = B // NW
CHUNK = 128
NCHUNK = RPW // CHUNK


def _tanh(x):
    xi = plsc.bitcast(x, jnp.int32)
    sign = xi & jnp.int32(-2147483648)
    a = plsc.bitcast(xi & jnp.int32(0x7FFFFFFF), jnp.float32)
    e = jnp.exp(-2.0 * a)
    u = e + 1.0
    w = 1.45710678 - 0.5 * u
    w = w * (2.0 - u * w)
    w = w * (2.0 - u * w)
    g = e * w
    th = 1.0 - (g + g)
    return plsc.bitcast(plsc.bitcast(th, jnp.int32) | sign, jnp.float32)


def _sqrt(x):
    i = plsc.bitcast(x, jnp.int32)
    y = plsc.bitcast(jnp.int32(0x5F3759DF) - (i >> 1), jnp.float32)
    for _ in range(3):
        y = y * (1.5 - 0.5 * x * y * y)
    return x * y


def _body(hq, rq, tq, hq_, rq_, tq_,
          hs, rs, ts, hs_, rs_, ts_,
          hl4, rl4, o1_hbm, o2_hbm,
          hq_v, rq_v, tq_v, hq2_v, rq2_v, tq2_v,
          hs_v, rs_v, ts_v, hs2_v, rs2_v, ts2_v,
          bh0, bh1, br0, br1, bt0, bt1,
          d1_v, d2_v, sem0, sem1, sems):
    wid = lax.axis_index("c") * NS + lax.axis_index("s")
    base = wid * RPW
    sl = pl.ds(base, RPW)

    stage = [
        pltpu.async_copy(hq.at[sl], hq_v, sems),
        pltpu.async_copy(rq.at[sl], rq_v, sems),
        pltpu.async_copy(tq.at[sl], tq_v, sems),
        pltpu.async_copy(hq_.at[sl], hq2_v, sems),
        pltpu.async_copy(rq_.at[sl], rq2_v, sems),
        pltpu.async_copy(tq_.at[sl], tq2_v, sems),
        pltpu.async_copy(hs.at[sl], hs_v, sems),
        pltpu.async_copy(rs.at[sl], rs_v, sems),
        pltpu.async_copy(ts.at[sl], ts_v, sems),
        pltpu.async_copy(hs_.at[sl], hs2_v, sems),
        pltpu.async_copy(rs_.at[sl], rs2_v, sems),
        pltpu.async_copy(ts_.at[sl], ts2_v, sems),
    ]
    for cp in stage:
        cp.wait()

    qrefs = ((hq_v, rq_v, tq_v), (hq2_v, rq2_v, tq2_v))
    srefs = ((hs_v, rs_v, ts_v), (hs2_v, rs2_v, ts2_v))
    bufs = ((bh0, bh1), (br0, br1), (bt0, bt1))
    tables = (hl4, rl4, hl4)
    dists = (d1_v, d2_v)
    sems2 = (sem0, sem1)

    def fire(k):
        trip, c = divmod(k, NCHUNK)
        bank = k & 1
        s = pl.ds(c * CHUNK, CHUNK)
        return [
            pltpu.async_copy(tables[i].at[qrefs[trip][i].at[s]],
                             bufs[i][bank], sems2[bank])
            for i in range(3)
        ]

    lanes = lax.iota(jnp.int32, 16)

    def compute(k):
        trip, c = divmod(k, NCHUNK)
        bank = k & 1
        hbuf, rbuf, tbuf = bufs[0][bank], bufs[1][bank], bufs[2][bank]
        hs_r, rs_r, ts_r = srefs[trip]
        dist_v = dists[trip]

        def group(g, _):
            off = c * CHUNK + g * 16
            rows = g * 16 + lanes
            ch = hs_r[pl.ds(off, 16)]
            cr = rs_r[pl.ds(off, 16)]
            ct = ts_r[pl.ds(off, 16)]
            z = jnp.zeros((16,), jnp.float32)

            def dim4(jj, acc):
                s_hh, s_rr, s_tt, s_ht, s_rth = acc
                j0 = jj * 4
                bh = ch + j0
                br = cr + j0
                bt = ct + j0
                for jo in range(4):
                    hv = _tanh(plsc.load_gather(hbuf, [rows, bh + jo]))
                    rv = _tanh(plsc.load_gather(rbuf, [rows, br + jo]))
                    tv = _tanh(plsc.load_gather(tbuf, [rows, bt + jo]))
                    s_hh = s_hh + hv * hv
                    s_rr = s_rr + rv * rv
                    s_tt = s_tt + tv * tv
                    s_ht = s_ht + hv * tv
                    s_rth = s_rth + rv * (tv - hv)
                return (s_hh, s_rr, s_tt, s_ht, s_rth)

            s_hh, s_rr, s_tt, s_ht, s_rth = lax.fori_loop(
                0, DIM // 4, dim4, (z, z, z, z, z))
            dist = (_sqrt(s_hh) + _sqrt(s_rr) + _sqrt(s_tt)
                    - 2.0 * (s_ht + s_rth))
            plsc.store_scatter(dist_v, [off + lanes], dist)
            return 0

        lax.fori_loop(0, CHUNK // 16, group, 0)

    pending = {0: fire(0)}
    for k in range(2 * NCHUNK):
        if k + 1 < 2 * NCHUNK:
            pending[k + 1] = fire(k + 1)
        for cp in pending.pop(k):
            cp.wait()
        compute(k)

    pltpu.sync_copy(d1_v, o1_hbm.at[sl])
    pltpu.sync_copy(d2_v, o2_hbm.at[sl])


@jax.jit
def kernel(h, r, t, h_, r_, t_, hl, rl):
    mesh = plsc.VectorSubcoreMesh(core_axis_name="c", subcore_axis_name="s")
    f = pl.kernel(
        _body,
        out_type=(jax.ShapeDtypeStruct((B,), jnp.float32),
                  jax.ShapeDtypeStruct((B,), jnp.float32)),
        mesh=mesh,
        compiler_params=pltpu.CompilerParams(needs_layout_passes=False),
        scratch_types=(
            [pltpu.VMEM((RPW,), jnp.int32)] * 12
            + [pltpu.VMEM((CHUNK, 128), jnp.float32)] * 6
            + [pltpu.VMEM((RPW,), jnp.float32)] * 2
            + [pltpu.SemaphoreType.DMA] * 3
        ),
    )
    idxs = [x.astype(jnp.int32) for x in (h, r, t, h_, r_, t_)]
    qs = [x >> 2 for x in idxs]
    ss = [(x & 3) << 5 for x in idxs]
    hl4 = hl.reshape(VOCAB // 4, 128)
    rl4 = rl.reshape(REL // 4, 128)
    return f(*qs, *ss, hl4, rl4)
